# baseline (speedup 1.0000x reference)
import functools

import jax
import jax.numpy as jnp
from jax import lax
from jax.experimental import pallas as pl
from jax.experimental.pallas import tpu as pltpu
from jax.experimental.pallas import tpu_sc as plsc

SEQ = 2048
EMBED = 768
VOCAB = 100000
VT = 1024


def _sc_gather(tok_emb, idx):
    info = plsc.get_sparse_core_info()
    nc, ns = info.num_cores, info.num_subcores
    nw = nc * ns
    b_per_w = SEQ // nw
    mesh = plsc.VectorSubcoreMesh(core_axis_name="c", subcore_axis_name="s")

    @functools.partial(
        pl.kernel,
        mesh=mesh,
        out_type=jax.ShapeDtypeStruct((SEQ, EMBED), jnp.float32),
        scratch_types=[
            pltpu.VMEM((b_per_w,), jnp.int32),
            pltpu.VMEM((b_per_w, EMBED), jnp.float32),
            pltpu.SemaphoreType.DMA,
        ],
    )
    def gather_kernel(tok_hbm, idx_hbm, x_hbm, idx_v, rows_v, sem):
        wid = lax.axis_index("s") * nc + lax.axis_index("c")
        base = wid * b_per_w
        pltpu.sync_copy(idx_hbm.at[pl.ds(base, b_per_w)], idx_v)
        pltpu.async_copy(tok_hbm.at[idx_v], rows_v, sem).wait()
        pltpu.sync_copy(rows_v, x_hbm.at[pl.ds(base, b_per_w)])

    return gather_kernel(tok_emb, idx)


def _mm_body(x_ref, p_ref, w_ref, o_ref):
    s = x_ref[...] + p_ref[...]
    o_ref[0] = lax.dot_general(
        s, w_ref[...],
        dimension_numbers=(((1,), (1,)), ((), ())),
        preferred_element_type=jnp.float32,
    )


def _tc_project(x, pos, W_out):
    grid = pl.cdiv(VOCAB, VT)
    return pl.pallas_call(
        _mm_body,
        grid=(grid,),
        in_specs=[
            pl.BlockSpec((SEQ, EMBED), lambda i: (0, 0)),
            pl.BlockSpec((SEQ, EMBED), lambda i: (0, 0)),
            pl.BlockSpec((VT, EMBED), lambda i: (i, 0)),
        ],
        out_specs=pl.BlockSpec((1, SEQ, VT), lambda i: (0, 0, i)),
        out_shape=jax.ShapeDtypeStruct((1, SEQ, VOCAB), jnp.float32),
    )(x, pos, W_out)


def kernel(in_idx, tok_emb, pos_emb, W_out):
    idx = in_idx.reshape(-1).astype(jnp.int32)
    x = _sc_gather(tok_emb, idx)
    return _tc_project(x, pos_emb[:SEQ], W_out)


# speedup vs baseline: 1.0007x; 1.0007x over previous
import functools

import jax
import jax.numpy as jnp
from jax import lax
from jax.experimental import pallas as pl
from jax.experimental.pallas import tpu as pltpu
from jax.experimental.pallas import tpu_sc as plsc

SEQ = 2048
EMBED = 768
VOCAB = 100000
VT = 1024


def _sc_gather(tok_emb, idx):
    info = plsc.get_sparse_core_info()
    nc, ns = info.num_cores, info.num_subcores
    nw = nc * ns
    b_per_w = SEQ // nw
    mesh = plsc.VectorSubcoreMesh(core_axis_name="c", subcore_axis_name="s")

    @functools.partial(
        pl.kernel,
        mesh=mesh,
        out_type=jax.ShapeDtypeStruct((SEQ, EMBED), jnp.float32),
        scratch_types=[
            pltpu.VMEM((b_per_w,), jnp.int32),
            pltpu.VMEM((b_per_w, EMBED), jnp.float32),
            pltpu.SemaphoreType.DMA,
        ],
    )
    def gather_kernel(tok_hbm, idx_hbm, x_hbm, idx_v, rows_v, sem):
        wid = lax.axis_index("s") * nc + lax.axis_index("c")
        base = wid * b_per_w
        pltpu.sync_copy(idx_hbm.at[pl.ds(base, b_per_w)], idx_v)
        pltpu.async_copy(tok_hbm.at[idx_v], rows_v, sem).wait()
        pltpu.sync_copy(rows_v, x_hbm.at[pl.ds(base, b_per_w)])

    return gather_kernel(tok_emb, idx)


def _mm_body(x_ref, p_ref, w_ref, o_ref, s_ref):
    @pl.when(pl.program_id(0) == 0)
    def _():
        s_ref[...] = x_ref[...] + p_ref[...]

    o_ref[0] = lax.dot_general(
        s_ref[...], w_ref[...],
        dimension_numbers=(((1,), (1,)), ((), ())),
        preferred_element_type=jnp.float32,
    )


def _tc_project(x, pos, W_out):
    grid = pl.cdiv(VOCAB, VT)
    return pl.pallas_call(
        _mm_body,
        grid=(grid,),
        in_specs=[
            pl.BlockSpec((SEQ, EMBED), lambda i: (0, 0)),
            pl.BlockSpec((SEQ, EMBED), lambda i: (0, 0)),
            pl.BlockSpec((VT, EMBED), lambda i: (i, 0)),
        ],
        out_specs=pl.BlockSpec((1, SEQ, VT), lambda i: (0, 0, i)),
        out_shape=jax.ShapeDtypeStruct((1, SEQ, VOCAB), jnp.float32),
        scratch_shapes=[pltpu.VMEM((SEQ, EMBED), jnp.float32)],
    )(x, pos, W_out)


def kernel(in_idx, tok_emb, pos_emb, W_out):
    idx = in_idx.reshape(-1).astype(jnp.int32)
    x = _sc_gather(tok_emb, idx)
    return _tc_project(x, pos_emb[:SEQ], W_out)


# speedup vs baseline: 2.7056x; 2.7038x over previous
import functools

import jax
import jax.numpy as jnp
from jax import lax
from jax.experimental import pallas as pl
from jax.experimental.pallas import tpu as pltpu
from jax.experimental.pallas import tpu_sc as plsc

SEQ = 2048
EMBED = 768
VOCAB = 100000
VT = 1024


def _sc_gather(tok_emb, idx):
    info = plsc.get_sparse_core_info()
    nc, ns = info.num_cores, info.num_subcores
    nw = nc * ns
    b_per_w = SEQ // nw
    mesh = plsc.VectorSubcoreMesh(core_axis_name="c", subcore_axis_name="s")

    @functools.partial(
        pl.kernel,
        mesh=mesh,
        out_type=jax.ShapeDtypeStruct((SEQ, EMBED), jnp.float32),
        scratch_types=[
            pltpu.VMEM((b_per_w,), jnp.int32),
            pltpu.VMEM((b_per_w, EMBED), jnp.float32),
            pltpu.SemaphoreType.DMA,
        ],
    )
    def gather_kernel(tok_hbm, idx_hbm, x_hbm, idx_v, rows_v, sem):
        wid = lax.axis_index("s") * nc + lax.axis_index("c")
        base = wid * b_per_w
        pltpu.sync_copy(idx_hbm.at[pl.ds(base, b_per_w)], idx_v)
        pltpu.async_copy(tok_hbm.at[idx_v], rows_v, sem).wait()
        pltpu.sync_copy(rows_v, x_hbm.at[pl.ds(base, b_per_w)])

    return gather_kernel(tok_emb, idx)


def _mm_body(x_ref, p_ref, w_ref, o_ref, st_ref):
    @pl.when(pl.program_id(0) == 0)
    def _():
        st_ref[...] = jnp.transpose(x_ref[...] + p_ref[...], (1, 0))

    o_ref[...] = lax.dot_general(
        w_ref[...], st_ref[...],
        dimension_numbers=(((1,), (0,)), ((), ())),
        preferred_element_type=jnp.float32,
    )


def _tc_project(x, pos, W_out):
    grid = pl.cdiv(VOCAB, VT)
    out_t = pl.pallas_call(
        _mm_body,
        grid=(grid,),
        in_specs=[
            pl.BlockSpec((SEQ, EMBED), lambda i: (0, 0)),
            pl.BlockSpec((SEQ, EMBED), lambda i: (0, 0)),
            pl.BlockSpec((VT, EMBED), lambda i: (i, 0)),
        ],
        out_specs=pl.BlockSpec((VT, SEQ), lambda i: (i, 0)),
        out_shape=jax.ShapeDtypeStruct((VOCAB, SEQ), jnp.float32),
        scratch_shapes=[pltpu.VMEM((EMBED, SEQ), jnp.float32)],
    )(x, pos, W_out)
    return jnp.transpose(out_t, (1, 0))[None]


def kernel(in_idx, tok_emb, pos_emb, W_out):
    idx = in_idx.reshape(-1).astype(jnp.int32)
    x = _sc_gather(tok_emb, idx)
    return _tc_project(x, pos_emb[:SEQ], W_out)
